# baseline (speedup 1.0000x reference)
import functools

import jax
import jax.numpy as jnp
from jax import lax
from jax.experimental import pallas as pl
from jax.experimental.pallas import tpu as pltpu
from jax.experimental.pallas import tpu_sc as plsc

N_STATE = 3
L = 16


@functools.partial(jax.jit, static_argnames=("B", "N", "F"))
def _sc_embed(sample_flat, embed_table, *, B, N, F):
    mesh = plsc.VectorSubcoreMesh(core_axis_name="c", subcore_axis_name="s")
    NW = mesh.num_cores * mesh.num_subcores
    assert B % NW == 0 and N % L == 0 and B % L == 0
    b_per_w = B // NW
    n_tok = b_per_w * N
    n_chunks = N // L
    R = N + 1
    n_first_grp = B // L

    @functools.partial(
        pl.kernel,
        mesh=mesh,
        out_type=[
            jax.ShapeDtypeStruct((B * R, F), jnp.float32),
            jax.ShapeDtypeStruct((B * R, F), jnp.float32),
            jax.ShapeDtypeStruct((B * N,), jnp.int32),
        ],
        scratch_types=[
            pltpu.VMEM((n_tok,), jnp.int32),
            pltpu.VMEM((n_tok,), jnp.int32),
            pltpu.VMEM((L, F), jnp.float32),
            pltpu.SemaphoreType.DMA,
            pltpu.SemaphoreType.DMA,
            pltpu.SemaphoreType.DMA,
        ],
    )
    def k(samp_hbm, table_hbm, dir_hbm, inv_hbm, tok_hbm,
          samp_v, tok_v, buf_v, sem_g, sem_d, sem_i):
        wid = lax.axis_index("s") * mesh.num_cores + lax.axis_index("c")
        b0 = wid * b_per_w
        iota = lax.iota(jnp.int32, L)

        tok_base = pl.multiple_of(wid * n_tok, n_tok)
        pltpu.sync_copy(samp_hbm.at[pl.ds(tok_base, n_tok)], samp_v)
        for m in range(n_tok // L):
            s = samp_v[pl.ds(L * m, L)]
            tok_v[pl.ds(L * m, L)] = (s + 1) >> 1
        pltpu.sync_copy(tok_v, tok_hbm.at[pl.ds(tok_base, n_tok)])

        first_grp = 2 * n_first_grp
        @pl.when(wid < first_grp)
        def _():
            pltpu.async_copy(
                table_hbm.at[jnp.full((L,), N_STATE, jnp.int32)],
                buf_v, sem_g).wait()
        for t in range(first_grp):
            dst = dir_hbm if t < n_first_grp else inv_hbm
            g = t % n_first_grp

            @pl.when(wid == t)
            def _():
                rows = R * (L * g + iota)
                pltpu.async_copy(buf_v, dst.at[rows], sem_d).wait()

        for r in range(b_per_w):
            base = (b0 + r) * R

            def chunk(c, _):
                p = pl.multiple_of(c * L, L)
                toks = tok_v[pl.ds(r * N + p, L)]
                pltpu.async_copy(table_hbm.at[toks], buf_v, sem_g).wait()
                cp_d = pltpu.async_copy(
                    buf_v, dir_hbm.at[base + 1 + p + iota], sem_d)
                cp_i = pltpu.async_copy(
                    buf_v, inv_hbm.at[base + N - p - iota], sem_i)
                cp_d.wait()
                cp_i.wait()
                return 0

            lax.fori_loop(0, n_chunks, chunk, 0)

    return k(sample_flat, embed_table)


def kernel(sample, embed_table, batch_size):
    B, N = sample.shape
    F = embed_table.shape[1]
    d, i, t = _sc_embed(sample.reshape(-1), embed_table, B=B, N=N, F=F)
    return (d.reshape(B, N + 1, F), i.reshape(B, N + 1, F), t.reshape(B, N))
